# baseline (speedup 1.0000x reference)
import functools

import jax
import jax.numpy as jnp
from jax import lax
from jax.experimental import pallas as pl
from jax.experimental.pallas import tpu as pltpu
from jax.experimental.pallas import tpu_sc as plsc

N_NODES = 10000
N_EDGES = 320000
DIM = 128
C = 128
TAIL = 16

NW = 32
E_PER_W = N_EDGES // NW
E_FULL = E_PER_W // C

NODE_BLK = 2000
NODE_VPAD = 32


def _edge_body(edge_idx, edge_table, edge_out,
               idx_e, buf0, buf1, buf2, tbuf, etab_sh,
               gsem0, gsem1, gsem2, ssem0, ssem1, ssem2, tsem):
    nc = 2
    sid = lax.axis_index("s")
    wid = sid * nc + lax.axis_index("c")

    @pl.when(sid == 0)
    def _():
        pltpu.sync_copy(edge_table, etab_sh)
    plsc.subcore_barrier()

    ebase = wid * E_PER_W
    pltpu.sync_copy(edge_idx.at[pl.ds(ebase, E_PER_W)], idx_e)

    bufs = (buf0, buf1, buf2)
    gsems = (gsem0, gsem1, gsem2)
    ssems = (ssem0, ssem1, ssem2)

    def g(j, k):
        pltpu.async_copy(etab_sh.at[idx_e.at[pl.ds(j * C, C)]],
                         bufs[k], gsems[k])

    def gwait(k):
        pltpu.make_async_copy(
            etab_sh.at[idx_e.at[pl.ds(0, C)]], bufs[k], gsems[k]).wait()

    def s(j, k):
        pltpu.async_copy(bufs[k], edge_out.at[pl.ds(ebase + j * C, C)],
                         ssems[k])

    def swait(k):
        pltpu.make_async_copy(
            bufs[k], edge_out.at[pl.ds(ebase, C)], ssems[k]).wait()

    nloop = E_FULL // 3

    def body(i, _):
        for k in range(3):
            c = 3 * i + k
            @pl.when(i > 0)
            def _():
                swait(k)
            g(c, k)
            k1 = (k - 1) % 3
            if k == 0:
                @pl.when(i > 0)
                def _():
                    gwait(k1)
                    s(c - 1, k1)
            else:
                gwait(k1)
                s(c - 1, k1)
        return 0

    lax.fori_loop(0, nloop, body, 0)

    gwait(2)
    s(E_FULL - 1, 2)
    pltpu.async_copy(
        etab_sh.at[idx_e.at[pl.ds(E_FULL * C, TAIL)]], tbuf, tsem).wait()
    pltpu.sync_copy(tbuf, edge_out.at[pl.ds(ebase + E_FULL * C, TAIL)])
    swait(0)
    swait(1)
    swait(2)


def _node_tc_body(idx_ref, tab_ref, out_ref):
    idx = idx_ref[0, 0, :]
    tab = tab_ref[...]
    idx2d = jnp.broadcast_to(idx[:, None], (NODE_BLK, DIM))
    acc = jnp.zeros((NODE_BLK, DIM), jnp.float32)
    for v in range(21):
        row = jnp.broadcast_to(tab[v][None, :], (NODE_BLK, DIM))
        acc = acc + jnp.where(idx2d == v, row, 0.0)
    out_ref[...] = acc


@jax.jit
def kernel(node_idx, edge_idx, node_table, edge_table):
    mesh = plsc.VectorSubcoreMesh(core_axis_name="c", subcore_axis_name="s")
    edge_fn = functools.partial(
        pl.kernel,
        out_type=jax.ShapeDtypeStruct((N_EDGES, DIM), jnp.float32),
        mesh=mesh,
        scratch_types=[
            pltpu.VMEM((E_PER_W,), jnp.int32),
            pltpu.VMEM((C, DIM), jnp.float32),
            pltpu.VMEM((C, DIM), jnp.float32),
            pltpu.VMEM((C, DIM), jnp.float32),
            pltpu.VMEM((TAIL, DIM), jnp.float32),
            pltpu.VMEM_SHARED((4, DIM), jnp.float32),
            pltpu.SemaphoreType.DMA,
            pltpu.SemaphoreType.DMA,
            pltpu.SemaphoreType.DMA,
            pltpu.SemaphoreType.DMA,
            pltpu.SemaphoreType.DMA,
            pltpu.SemaphoreType.DMA,
            pltpu.SemaphoreType.DMA,
        ],
    )(_edge_body)
    edge_emb = edge_fn(edge_idx, edge_table)

    nb = N_NODES // NODE_BLK
    ntab = jnp.zeros((NODE_VPAD, DIM), jnp.float32).at[:21].set(node_table)
    node_emb = pl.pallas_call(
        _node_tc_body,
        grid=(nb,),
        in_specs=[
            pl.BlockSpec((1, 1, NODE_BLK), lambda i: (i, 0, 0)),
            pl.BlockSpec((NODE_VPAD, DIM), lambda i: (0, 0)),
        ],
        out_specs=pl.BlockSpec((NODE_BLK, DIM), lambda i: (i, 0)),
        out_shape=jax.ShapeDtypeStruct((N_NODES, DIM), jnp.float32),
    )(node_idx.reshape(nb, 1, NODE_BLK), ntab)

    return (node_emb, edge_emb)


# speedup vs baseline: 1.0103x; 1.0103x over previous
import functools

import jax
import jax.numpy as jnp
from jax import lax
from jax.experimental import pallas as pl
from jax.experimental.pallas import tpu as pltpu
from jax.experimental.pallas import tpu_sc as plsc

N_NODES = 10000
N_EDGES = 320000
DIM = 128
C = 128
TAIL = 16

NW = 32
E_PER_W = N_EDGES // NW
E_FULL = E_PER_W // C

NODE_BLK = 2000
NODE_VPAD = 32


def _edge_body(edge_idx, edge_table, edge_out,
               idx_e, buf0, buf1, tbuf, etab_sh,
               gsem0, gsem1, ssem0, ssem1, tsem):
    nc = 2
    sid = lax.axis_index("s")
    wid = sid * nc + lax.axis_index("c")

    @pl.when(sid == 0)
    def _():
        pltpu.sync_copy(edge_table, etab_sh)
    plsc.subcore_barrier()

    ebase = wid * E_PER_W
    pltpu.sync_copy(edge_idx.at[pl.ds(ebase, E_PER_W)], idx_e)

    def g(j, buf, sem):
        pltpu.async_copy(etab_sh.at[idx_e.at[pl.ds(j * C, C)]], buf, sem)

    def gwait(buf, sem):
        pltpu.make_async_copy(
            etab_sh.at[idx_e.at[pl.ds(0, C)]], buf, sem).wait()

    def s(j, buf, sem):
        pltpu.async_copy(buf, edge_out.at[pl.ds(ebase + j * C, C)], sem)

    def swait(buf, sem):
        pltpu.make_async_copy(buf, edge_out.at[pl.ds(ebase, C)], sem).wait()

    nloop = E_FULL // 2
    g(0, buf0, gsem0)

    def body(i, _):
        c0 = 2 * i
        gwait(buf0, gsem0)
        s(c0, buf0, ssem0)

        @pl.when(i > 0)
        def _():
            swait(buf1, ssem1)
        g(c0 + 1, buf1, gsem1)
        gwait(buf1, gsem1)
        s(c0 + 1, buf1, ssem1)
        swait(buf0, ssem0)

        @pl.when(i < nloop - 1)
        def _():
            g(c0 + 2, buf0, gsem0)
        return 0

    lax.fori_loop(0, nloop, body, 0)

    pltpu.async_copy(
        etab_sh.at[idx_e.at[pl.ds(E_FULL * C, TAIL)]], tbuf, tsem).wait()
    pltpu.sync_copy(tbuf, edge_out.at[pl.ds(ebase + E_FULL * C, TAIL)])
    swait(buf1, ssem1)


def _node_tc_body(idx_ref, tab_ref, out_ref):
    idx = idx_ref[0, 0, :]
    tab = tab_ref[...]
    idx2d = jnp.broadcast_to(idx[:, None], (NODE_BLK, DIM))
    acc = jnp.zeros((NODE_BLK, DIM), jnp.float32)
    for v in range(21):
        row = jnp.broadcast_to(tab[v][None, :], (NODE_BLK, DIM))
        acc = acc + jnp.where(idx2d == v, row, 0.0)
    out_ref[...] = acc


@jax.jit
def kernel(node_idx, edge_idx, node_table, edge_table):
    mesh = plsc.VectorSubcoreMesh(core_axis_name="c", subcore_axis_name="s")
    edge_fn = functools.partial(
        pl.kernel,
        out_type=jax.ShapeDtypeStruct((N_EDGES, DIM), jnp.float32),
        mesh=mesh,
        scratch_types=[
            pltpu.VMEM((E_PER_W,), jnp.int32),
            pltpu.VMEM((C, DIM), jnp.float32),
            pltpu.VMEM((C, DIM), jnp.float32),
            pltpu.VMEM((TAIL, DIM), jnp.float32),
            pltpu.VMEM_SHARED((4, DIM), jnp.float32),
            pltpu.SemaphoreType.DMA,
            pltpu.SemaphoreType.DMA,
            pltpu.SemaphoreType.DMA,
            pltpu.SemaphoreType.DMA,
            pltpu.SemaphoreType.DMA,
        ],
    )(_edge_body)
    edge_emb = edge_fn(edge_idx, edge_table)

    nb = N_NODES // NODE_BLK
    ntab = jnp.zeros((NODE_VPAD, DIM), jnp.float32).at[:21].set(node_table)
    node_emb = pl.pallas_call(
        _node_tc_body,
        grid=(nb,),
        in_specs=[
            pl.BlockSpec((1, 1, NODE_BLK), lambda i: (i, 0, 0)),
            pl.BlockSpec((NODE_VPAD, DIM), lambda i: (0, 0)),
        ],
        out_specs=pl.BlockSpec((NODE_BLK, DIM), lambda i: (i, 0)),
        out_shape=jax.ShapeDtypeStruct((N_NODES, DIM), jnp.float32),
    )(node_idx.reshape(nb, 1, NODE_BLK), ntab)

    return (node_emb, edge_emb)
